# Initial kernel scaffold; baseline (speedup 1.0000x reference)
import jax
import jax.numpy as jnp
from jax.experimental import pallas as pl


def kernel(x, edge_index, edge_attr, batch, score_w1, score_b1, dist_w1, dist_b1, score_w2, score_b2, dist_w2, dist_b2):
    raise NotImplementedError("write your pallas kernel here")


# speedup vs baseline: 20.6634x; 20.6634x over previous
import functools

import jax
import jax.numpy as jnp
from jax import lax
from jax.experimental import pallas as pl
from jax.experimental.pallas import tpu as pltpu
from jax.experimental.pallas import tpu_sc as plsc

G = 64
RATIO_K = 0.5
TN = 128
NC = 2
NS = 16
NW = NC * NS


def _k1_body(x_ref, sw1_ref, sw2_ref, sb1_ref, xs1_ref, xsw2_ref):
    xb = x_ref[...]
    xs1_ref[...] = jnp.dot(xb, sw1_ref[...],
                           preferred_element_type=jnp.float32) + sb1_ref[0]
    xsw2_ref[...] = jnp.dot(xb, sw2_ref[...],
                            preferred_element_type=jnp.float32)


def _k2_body(ea_ref, dw1_ref, dw2_ref, db1_ref, db2_ref, w1_ref, w2_ref):
    eb = ea_ref[...]
    w1_ref[...] = jnp.dot(eb, dw1_ref[...],
                          preferred_element_type=jnp.float32) + db1_ref[0]
    w2_ref[...] = jnp.dot(eb, dw2_ref[...],
                          preferred_element_type=jnp.float32) + db2_ref[0]


def _sc_edge_body(epw, npad, e0_hbm, e1_hbm, w_hbm, a_hbm, b_hbm,
                  agg_out, cnt_out, e0_v, e1_v, w_v, a_v, b_v, agg_v, cnt_v):
    wid = lax.axis_index("s") * NC + lax.axis_index("c")
    base = wid * epw
    pltpu.sync_copy(e0_hbm.at[pl.ds(base, epw)], e0_v)
    pltpu.sync_copy(e1_hbm.at[pl.ds(base, epw)], e1_v)
    pltpu.sync_copy(w_hbm.at[pl.ds(base, epw)], w_v)
    pltpu.sync_copy(a_hbm, a_v)
    pltpu.sync_copy(b_hbm, b_v)

    zeros = jnp.zeros((16,), jnp.float32)

    def zbody(i, carry):
        agg_v[pl.ds(i * 16, 16)] = zeros
        cnt_v[pl.ds(i * 16, 16)] = zeros
        return carry

    lax.fori_loop(0, npad // 16, zbody, 0, unroll=4)

    def body(i, carry):
        sl = pl.ds(i * 16, 16)
        e0s = e0_v[sl]
        e1s = e1_v[sl]
        ws = w_v[sl]
        av = plsc.load_gather(a_v, [e0s])
        bv = plsc.load_gather(b_v, [e0s])
        plsc.addupdate_scatter(agg_v, [e1s], av * ws)
        plsc.addupdate_scatter(cnt_v, [e1s], bv)
        return carry

    lax.fori_loop(0, epw // 16, body, 0, unroll=4)

    pltpu.sync_copy(agg_v, agg_out.at[wid])
    pltpu.sync_copy(cnt_v, cnt_out.at[wid])


def _make_sc_edge(epw, npad):
    mesh = plsc.VectorSubcoreMesh(core_axis_name="c", subcore_axis_name="s",
                                  num_cores=NC, num_subcores=NS)
    return pl.kernel(
        functools.partial(_sc_edge_body, epw, npad),
        out_type=[jax.ShapeDtypeStruct((NW, npad), jnp.float32),
                  jax.ShapeDtypeStruct((NW, npad), jnp.float32)],
        mesh=mesh,
        compiler_params=pltpu.CompilerParams(needs_layout_passes=False),
        scratch_types=[pltpu.VMEM((epw,), jnp.int32),
                       pltpu.VMEM((epw,), jnp.int32),
                       pltpu.VMEM((epw,), jnp.float32),
                       pltpu.VMEM((npad,), jnp.float32),
                       pltpu.VMEM((npad,), jnp.float32),
                       pltpu.VMEM((npad,), jnp.float32),
                       pltpu.VMEM((npad,), jnp.float32)],
    )


def _r_body(nb,
            pagg_ref, pcnt_ref, sec_ref, alive_ref, batchf_ref, xsw2_ref,
            nal_ref, sb2_ref, lo_ref, hi_ref,
            s_ref, sel_ref, an_ref, bn_ref, cg_ref,
            colS, colSec, colCode, rowS, cgacc):
    ph = pl.program_id(0)
    i = pl.program_id(1)

    row = pl.ds(i, 1)

    @pl.when(ph == 0)
    def _phase0():
        agg = jnp.sum(pagg_ref[:, row, :], axis=0)
        cnt = jnp.sum(pcnt_ref[:, row, :], axis=0)
        s_row = agg / jnp.maximum(cnt, 1.0)
        s_ref[row, :] = s_row
        rowS[row, :] = s_row
        sl = pl.ds(i * TN, TN)
        colS[sl, :] = s_row.reshape(TN, 1)
        colSec[sl, :] = sec_ref[row, :].reshape(TN, 1)
        code = jnp.where(alive_ref[row, :] > 0.0, batchf_ref[row, :], -1.0)
        colCode[sl, :] = code.reshape(TN, 1)

    @pl.when(ph == 1)
    def _phase1():
        s_row = rowS[row, :]
        sec_row = sec_ref[row, :]
        b_row = batchf_ref[row, :]
        gp = ((i * TN).astype(jnp.float32)
              + lax.broadcasted_iota(jnp.int32, (1, TN), 1).astype(jnp.float32))

        def body(j, rank):
            sl = pl.ds(j * TN, TN)
            sqc = colS[sl, :]
            secqc = colSec[sl, :]
            codeqc = colCode[sl, :]
            gq = ((j * TN).astype(jnp.float32)
                  + lax.broadcasted_iota(jnp.int32, (TN, 1), 0).astype(jnp.float32))
            same = codeqc == b_row
            ahead = ((sqc > s_row)
                     | ((sqc == s_row)
                        & ((secqc > sec_row)
                           | ((secqc == sec_row) & (gq < gp)))))
            hit = jnp.where(same & ahead, 1.0, 0.0)
            return rank + jnp.sum(hit, axis=0, keepdims=True)

        lo = lo_ref[i]
        hi = hi_ref[i]
        rank = lax.fori_loop(lo, hi + 1, body,
                             jnp.zeros((1, TN), jnp.float32))

        kcol = jnp.ceil(RATIO_K * nal_ref[...])
        gio = lax.broadcasted_iota(jnp.int32, (G, 1), 0).astype(jnp.float32)
        ohm = gio == b_row
        kp = jnp.sum(jnp.where(ohm, kcol, 0.0), axis=0, keepdims=True)
        selr = jnp.where((alive_ref[row, :] > 0.0) & (rank < kp), 1.0, 0.0)
        sel_ref[row, :] = selr
        t1 = jnp.tanh(s_row)
        an_ref[row, :] = selr * (t1 * xsw2_ref[row, :] + sb2_ref[0])
        bn_ref[row, :] = selr

        prev = jnp.where(i == 0, jnp.zeros((G, 1), jnp.float32), cgacc[...])
        new = prev + jnp.sum(jnp.where(ohm, selr, 0.0), axis=1, keepdims=True)
        cgacc[...] = new
        cg_ref[...] = new


def _make_r(nb, npad):
    grid = (2, nb)
    full_row = pl.BlockSpec((nb, TN), lambda ph, i: (0, 0))
    return pl.pallas_call(
        functools.partial(_r_body, nb),
        grid=grid,
        in_specs=[
            pl.BlockSpec((NW, nb, TN), lambda ph, i: (0, 0, 0)),
            pl.BlockSpec((NW, nb, TN), lambda ph, i: (0, 0, 0)),
            full_row,
            full_row,
            full_row,
            full_row,
            pl.BlockSpec((G, 1), lambda ph, i: (0, 0)),
            pl.BlockSpec(memory_space=pltpu.SMEM),
            pl.BlockSpec(memory_space=pltpu.SMEM),
            pl.BlockSpec(memory_space=pltpu.SMEM),
        ],
        out_specs=[
            full_row,
            full_row,
            full_row,
            full_row,
            pl.BlockSpec((G, 1), lambda ph, i: (0, 0)),
        ],
        out_shape=[
            jax.ShapeDtypeStruct((nb, TN), jnp.float32),
            jax.ShapeDtypeStruct((nb, TN), jnp.float32),
            jax.ShapeDtypeStruct((nb, TN), jnp.float32),
            jax.ShapeDtypeStruct((nb, TN), jnp.float32),
            jax.ShapeDtypeStruct((G, 1), jnp.float32),
        ],
        scratch_shapes=[
            pltpu.VMEM((nb * TN, 1), jnp.float32),
            pltpu.VMEM((nb * TN, 1), jnp.float32),
            pltpu.VMEM((nb * TN, 1), jnp.float32),
            pltpu.VMEM((nb, TN), jnp.float32),
            pltpu.VMEM((G, 1), jnp.float32),
        ],
    )


def _f_body(nb, x_ref, batchf_ref, s1_ref, s2_ref, sel1_ref, sel2_ref,
            cnt0_ref, c1_ref, c2_ref, out_ref, acc):
    i = pl.program_id(0)

    @pl.when(i == 0)
    def _init():
        acc[...] = jnp.zeros_like(acc)

    row = pl.ds(i, 1)
    t1 = jnp.tanh(s1_ref[row, :])
    t2 = jnp.tanh(s2_ref[row, :])
    ic0 = 1.0 / jnp.maximum(cnt0_ref[...], 1.0)
    ic1 = 1.0 / jnp.maximum(c1_ref[...], 1.0)
    ic2 = 1.0 / jnp.maximum(c2_ref[...], 1.0)
    gio = lax.broadcasted_iota(jnp.int32, (G, 1), 0).astype(jnp.float32)
    ohm = gio == batchf_ref[row, :]
    w1r = sel1_ref[row, :] * t1
    w2r = sel2_ref[row, :] * t1 * t2
    A = jnp.where(ohm, ic0 + w1r * ic1 + w2r * ic2, 0.0)
    acc[...] += jnp.dot(A, x_ref[...], preferred_element_type=jnp.float32)

    @pl.when(i == nb - 1)
    def _fin():
        out_ref[...] = acc[...]


def _make_f(nb, npad, d):
    full_row = pl.BlockSpec((nb, TN), lambda i: (0, 0))
    gcol = pl.BlockSpec((G, 1), lambda i: (0, 0))
    return pl.pallas_call(
        functools.partial(_f_body, nb),
        grid=(nb,),
        in_specs=[pl.BlockSpec((TN, d), lambda i: (i, 0)),
                  full_row, full_row, full_row, full_row, full_row,
                  gcol, gcol, gcol],
        out_specs=pl.BlockSpec((G, d), lambda i: (0, 0)),
        out_shape=jax.ShapeDtypeStruct((G, d), jnp.float32),
        scratch_shapes=[pltpu.VMEM((G, d), jnp.float32)],
    )


def kernel(x, edge_index, edge_attr, batch, score_w1, score_b1, dist_w1,
           dist_b1, score_w2, score_b2, dist_w2, dist_b2):
    N, D = x.shape
    E = edge_attr.shape[0]
    DE = edge_attr.shape[1]

    nb = -(-N // TN)
    npad = nb * TN
    be = 2560
    ep = -(-E // be) * be
    epw = ep // NW

    xp = jnp.pad(x, ((0, npad - N), (0, 0)))
    batch_p = jnp.pad(batch.astype(jnp.int32), (0, npad - N),
                      constant_values=G - 1)
    batchf = batch_p.astype(jnp.float32).reshape(nb, TN)
    e0 = jnp.pad(edge_index[0].astype(jnp.int32), (0, ep - E))
    e1 = jnp.pad(edge_index[1].astype(jnp.int32), (0, ep - E),
                 constant_values=npad - 1)
    eap = jnp.pad(edge_attr, ((0, ep - E), (0, 0)))

    k1 = pl.pallas_call(
        _k1_body, grid=(nb,),
        in_specs=[pl.BlockSpec((TN, D), lambda i: (i, 0)),
                  pl.BlockSpec((D, 1), lambda i: (0, 0)),
                  pl.BlockSpec((D, 1), lambda i: (0, 0)),
                  pl.BlockSpec(memory_space=pltpu.SMEM)],
        out_specs=[pl.BlockSpec((TN, 1), lambda i: (i, 0)),
                   pl.BlockSpec((TN, 1), lambda i: (i, 0))],
        out_shape=[jax.ShapeDtypeStruct((npad, 1), jnp.float32),
                   jax.ShapeDtypeStruct((npad, 1), jnp.float32)],
    )
    xs1_c, xsw2_c = k1(xp, score_w1, score_w2, score_b1)

    eb_n = ep // be
    k2 = pl.pallas_call(
        _k2_body, grid=(eb_n,),
        in_specs=[pl.BlockSpec((be, DE), lambda i: (i, 0)),
                  pl.BlockSpec((DE, 1), lambda i: (0, 0)),
                  pl.BlockSpec((DE, 1), lambda i: (0, 0)),
                  pl.BlockSpec(memory_space=pltpu.SMEM),
                  pl.BlockSpec(memory_space=pltpu.SMEM)],
        out_specs=[pl.BlockSpec((be, 1), lambda i: (i, 0)),
                   pl.BlockSpec((be, 1), lambda i: (i, 0))],
        out_shape=[jax.ShapeDtypeStruct((ep, 1), jnp.float32),
                   jax.ShapeDtypeStruct((ep, 1), jnp.float32)],
    )
    w1_c, w2_c = k2(eap, dist_w1, dist_w2, dist_b1, dist_b2)
    w1 = w1_c.reshape(ep)
    w2 = w2_c.reshape(ep)

    gidx = jnp.arange(G, dtype=batch_p.dtype)
    starts = jnp.searchsorted(batch_p[:N], gidx, side="left").astype(jnp.int32)
    ends = jnp.searchsorted(batch_p[:N], gidx, side="right").astype(jnp.int32)
    counts = (ends - starts).astype(jnp.float32).reshape(G, 1)
    tile0 = jnp.arange(nb, dtype=jnp.int32) * TN
    first_node = jnp.minimum(tile0, N - 1)
    last_node = jnp.minimum(tile0 + TN - 1, N - 1)
    g_first = batch_p[first_node]
    g_last = batch_p[last_node]
    lo = starts[g_first] // TN
    hi = jnp.maximum(ends[g_last] - 1, 0) // TN

    sc_edge = _make_sc_edge(epw, npad)
    r_call = _make_r(nb, npad)

    a1 = xs1_c.reshape(npad)
    b1 = jnp.ones((npad,), jnp.float32)
    agg1, cnt1 = sc_edge(e0, e1, w1, a1, b1)
    zeros_row = jnp.zeros((nb, TN), jnp.float32)
    alive1 = (jnp.arange(npad, dtype=jnp.int32) < N).astype(
        jnp.float32).reshape(nb, TN)
    xsw2_r = xsw2_c.reshape(nb, TN)
    s1, sel1, a2r, b2r, c1 = r_call(
        agg1.reshape(NW, nb, TN), cnt1.reshape(NW, nb, TN),
        zeros_row, alive1, batchf, xsw2_r, counts, score_b2, lo, hi)

    agg2, cnt2 = sc_edge(e0, e1, w2, a2r.reshape(npad), b2r.reshape(npad))
    s2, sel2, _, _, c2 = r_call(
        agg2.reshape(NW, nb, TN), cnt2.reshape(NW, nb, TN),
        s1, sel1, batchf, xsw2_r, c1, score_b2, lo, hi)

    f_call = _make_f(nb, npad, D)
    out = f_call(xp, batchf, s1, s2, sel1, sel2, counts, c1, c2)
    return out


# speedup vs baseline: 34.4599x; 1.6677x over previous
import functools

import jax
import jax.numpy as jnp
from jax import lax
from jax.experimental import pallas as pl
from jax.experimental.pallas import tpu as pltpu
from jax.experimental.pallas import tpu_sc as plsc

G = 64
RATIO_K = 0.5
TN = 128
NC = 2
NS = 16
NW = NC * NS


def _k1_body(x_ref, sw1_ref, sw2_ref, sb1_ref, xs1_ref, xsw2_ref):
    i = pl.program_id(0)
    xb = x_ref[...]
    dn = (((0,), (1,)), ((), ()))
    c1 = lax.dot_general(sw1_ref[...], xb, dn,
                         preferred_element_type=jnp.float32) + sb1_ref[0]
    c2 = lax.dot_general(sw2_ref[...], xb, dn,
                         preferred_element_type=jnp.float32)
    xs1_ref[pl.ds(i, 1), :] = c1
    xsw2_ref[pl.ds(i, 1), :] = c2


def _k2_body(eaT_ref, dw1_ref, dw2_ref, db1_ref, db2_ref, w1_ref, w2_ref):
    blk = eaT_ref[...]
    w1_ref[...] = (jnp.sum(blk * dw1_ref[...], axis=0, keepdims=True)
                   + db1_ref[0])
    w2_ref[...] = (jnp.sum(blk * dw2_ref[...], axis=0, keepdims=True)
                   + db2_ref[0])


def _sc_edge_body(epw, npad, ei_hbm, w_hbm, a_hbm, b_hbm,
                  agg_out, cnt_out, ei_v, w_v, a_v, b_v, agg_v, cnt_v):
    wid = lax.axis_index("s") * NC + lax.axis_index("c")
    base = wid * epw
    pltpu.sync_copy(ei_hbm.at[:, pl.ds(base, epw)], ei_v)
    pltpu.sync_copy(w_hbm.at[pl.ds(base, epw)], w_v)
    pltpu.sync_copy(a_hbm, a_v)
    pltpu.sync_copy(b_hbm, b_v)

    zeros = jnp.zeros((16,), jnp.float32)

    def zbody(i, carry):
        agg_v[pl.ds(i * 16, 16)] = zeros
        cnt_v[pl.ds(i * 16, 16)] = zeros
        return carry

    lax.fori_loop(0, npad // 16, zbody, 0, unroll=4)

    def body(i, carry):
        sl = pl.ds(i * 16, 16)
        e0s = ei_v[0, sl]
        e1s = ei_v[1, sl]
        ws = w_v[sl]
        av = plsc.load_gather(a_v, [e0s])
        bv = plsc.load_gather(b_v, [e0s])
        plsc.addupdate_scatter(agg_v, [e1s], av * ws)
        plsc.addupdate_scatter(cnt_v, [e1s], bv)
        return carry

    lax.fori_loop(0, epw // 16, body, 0, unroll=4)

    pltpu.sync_copy(agg_v, agg_out.at[wid])
    pltpu.sync_copy(cnt_v, cnt_out.at[wid])


def _make_sc_edge(epw, npad):
    mesh = plsc.VectorSubcoreMesh(core_axis_name="c", subcore_axis_name="s",
                                  num_cores=NC, num_subcores=NS)
    return pl.kernel(
        functools.partial(_sc_edge_body, epw, npad),
        out_type=[jax.ShapeDtypeStruct((NW, npad), jnp.float32),
                  jax.ShapeDtypeStruct((NW, npad), jnp.float32)],
        mesh=mesh,
        compiler_params=pltpu.CompilerParams(needs_layout_passes=False),
        scratch_types=[pltpu.VMEM((2, epw), jnp.int32),
                       pltpu.VMEM((epw,), jnp.float32),
                       pltpu.VMEM((npad,), jnp.float32),
                       pltpu.VMEM((npad,), jnp.float32),
                       pltpu.VMEM((npad,), jnp.float32),
                       pltpu.VMEM((npad,), jnp.float32)],
    )


def _r_body(nb,
            pagg_ref, pcnt_ref, sec_ref, alive_ref, batchf_ref, xsw2_ref,
            nal_ref, sb2_ref, lo_ref, hi_ref,
            s_ref, sel_ref, an_ref, bn_ref, cg_ref,
            colS, colSec, colCode, rowS, cgacc):
    ph = pl.program_id(0)
    i = pl.program_id(1)

    row = pl.ds(i, 1)

    @pl.when(ph == 0)
    def _phase0():
        agg = jnp.sum(pagg_ref[:, row, :], axis=0)
        cnt = jnp.sum(pcnt_ref[:, row, :], axis=0)
        s_row = agg / jnp.maximum(cnt, 1.0)
        s_ref[row, :] = s_row
        rowS[row, :] = s_row
        sl = pl.ds(i * TN, TN)
        colS[sl, :] = s_row.reshape(TN, 1)
        colSec[sl, :] = sec_ref[row, :].reshape(TN, 1)
        code = jnp.where(alive_ref[row, :] > 0.0, batchf_ref[row, :], -1.0)
        colCode[sl, :] = code.reshape(TN, 1)

    @pl.when(ph == 1)
    def _phase1():
        s_row = rowS[row, :]
        sec_row = sec_ref[row, :]
        b_row = batchf_ref[row, :]
        gp = ((i * TN).astype(jnp.float32)
              + lax.broadcasted_iota(jnp.int32, (1, TN), 1).astype(jnp.float32))

        def body(j, rank):
            sl = pl.ds(j * TN, TN)
            sqc = colS[sl, :]
            secqc = colSec[sl, :]
            codeqc = colCode[sl, :]
            gq = ((j * TN).astype(jnp.float32)
                  + lax.broadcasted_iota(jnp.int32, (TN, 1), 0).astype(jnp.float32))
            same = codeqc == b_row
            ahead = ((sqc > s_row)
                     | ((sqc == s_row)
                        & ((secqc > sec_row)
                           | ((secqc == sec_row) & (gq < gp)))))
            hit = jnp.where(same & ahead, 1.0, 0.0)
            return rank + jnp.sum(hit, axis=0, keepdims=True)

        lo = lo_ref[i]
        hi = hi_ref[i]
        rank = lax.fori_loop(lo, hi + 1, body,
                             jnp.zeros((1, TN), jnp.float32))

        kcol = jnp.ceil(RATIO_K * nal_ref[...])
        gio = lax.broadcasted_iota(jnp.int32, (G, 1), 0).astype(jnp.float32)
        ohm = gio == b_row
        kp = jnp.sum(jnp.where(ohm, kcol, 0.0), axis=0, keepdims=True)
        selr = jnp.where((alive_ref[row, :] > 0.0) & (rank < kp), 1.0, 0.0)
        sel_ref[row, :] = selr
        t1 = jnp.tanh(s_row)
        an_ref[row, :] = selr * (t1 * xsw2_ref[row, :] + sb2_ref[0])
        bn_ref[row, :] = selr

        prev = jnp.where(i == 0, jnp.zeros((G, 1), jnp.float32), cgacc[...])
        new = prev + jnp.sum(jnp.where(ohm, selr, 0.0), axis=1, keepdims=True)
        cgacc[...] = new
        cg_ref[...] = new


def _make_r(nb, npad):
    grid = (2, nb)
    full_row = pl.BlockSpec((nb, TN), lambda ph, i: (0, 0))
    return pl.pallas_call(
        functools.partial(_r_body, nb),
        grid=grid,
        in_specs=[
            pl.BlockSpec((NW, nb, TN), lambda ph, i: (0, 0, 0)),
            pl.BlockSpec((NW, nb, TN), lambda ph, i: (0, 0, 0)),
            full_row,
            full_row,
            full_row,
            full_row,
            pl.BlockSpec((G, 1), lambda ph, i: (0, 0)),
            pl.BlockSpec(memory_space=pltpu.SMEM),
            pl.BlockSpec(memory_space=pltpu.SMEM),
            pl.BlockSpec(memory_space=pltpu.SMEM),
        ],
        out_specs=[
            full_row,
            full_row,
            full_row,
            full_row,
            pl.BlockSpec((G, 1), lambda ph, i: (0, 0)),
        ],
        out_shape=[
            jax.ShapeDtypeStruct((nb, TN), jnp.float32),
            jax.ShapeDtypeStruct((nb, TN), jnp.float32),
            jax.ShapeDtypeStruct((nb, TN), jnp.float32),
            jax.ShapeDtypeStruct((nb, TN), jnp.float32),
            jax.ShapeDtypeStruct((G, 1), jnp.float32),
        ],
        scratch_shapes=[
            pltpu.VMEM((nb * TN, 1), jnp.float32),
            pltpu.VMEM((nb * TN, 1), jnp.float32),
            pltpu.VMEM((nb * TN, 1), jnp.float32),
            pltpu.VMEM((nb, TN), jnp.float32),
            pltpu.VMEM((G, 1), jnp.float32),
        ],
    )


def _f_body(nb, x_ref, batchf_ref, s1_ref, s2_ref, sel1_ref, sel2_ref,
            cnt0_ref, c1_ref, c2_ref, out_ref, acc):
    i = pl.program_id(0)

    @pl.when(i == 0)
    def _init():
        acc[...] = jnp.zeros_like(acc)

    row = pl.ds(i, 1)
    t1 = jnp.tanh(s1_ref[row, :])
    t2 = jnp.tanh(s2_ref[row, :])
    ic0 = 1.0 / jnp.maximum(cnt0_ref[...], 1.0)
    ic1 = 1.0 / jnp.maximum(c1_ref[...], 1.0)
    ic2 = 1.0 / jnp.maximum(c2_ref[...], 1.0)
    gio = lax.broadcasted_iota(jnp.int32, (G, 1), 0).astype(jnp.float32)
    ohm = gio == batchf_ref[row, :]
    w1r = sel1_ref[row, :] * t1
    w2r = sel2_ref[row, :] * t1 * t2
    A = jnp.where(ohm, ic0 + w1r * ic1 + w2r * ic2, 0.0)
    acc[...] += jnp.dot(A, x_ref[...], preferred_element_type=jnp.float32)

    @pl.when(i == nb - 1)
    def _fin():
        out_ref[...] = acc[...]


def _make_f(nb, npad, d):
    full_row = pl.BlockSpec((nb, TN), lambda i: (0, 0))
    gcol = pl.BlockSpec((G, 1), lambda i: (0, 0))
    return pl.pallas_call(
        functools.partial(_f_body, nb),
        grid=(nb,),
        in_specs=[pl.BlockSpec((TN, d), lambda i: (i, 0)),
                  full_row, full_row, full_row, full_row, full_row,
                  gcol, gcol, gcol],
        out_specs=pl.BlockSpec((G, d), lambda i: (0, 0)),
        out_shape=jax.ShapeDtypeStruct((G, d), jnp.float32),
        scratch_shapes=[pltpu.VMEM((G, d), jnp.float32)],
    )


def kernel(x, edge_index, edge_attr, batch, score_w1, score_b1, dist_w1,
           dist_b1, score_w2, score_b2, dist_w2, dist_b2):
    N, D = x.shape
    E = edge_attr.shape[0]
    DE = edge_attr.shape[1]

    nb = -(-N // TN)
    npad = nb * TN
    be = 2560
    ek = -(-E // be) * be
    ep = -(-E // 20480) * 20480
    epw = ep // NW

    xp = jnp.pad(x, ((0, npad - N), (0, 0)))
    batch_p = jnp.pad(batch.astype(jnp.int32), (0, npad - N),
                      constant_values=G - 1)
    batchf = batch_p.astype(jnp.float32).reshape(nb, TN)
    ei = edge_index.astype(jnp.int32)
    eaT = edge_attr.T
    if ep != E:
        pad1 = jnp.full((1, ep - E), 0, jnp.int32)
        pad2 = jnp.full((1, ep - E), npad - 1, jnp.int32)
        ei = jnp.concatenate(
            [ei, jnp.concatenate([pad1, pad2], axis=0)], axis=1)
    if ek != E:
        eaT = jnp.pad(eaT, ((0, 0), (0, ek - E)))

    k1 = pl.pallas_call(
        _k1_body, grid=(nb,),
        in_specs=[pl.BlockSpec((TN, D), lambda i: (i, 0)),
                  pl.BlockSpec((D, 1), lambda i: (0, 0)),
                  pl.BlockSpec((D, 1), lambda i: (0, 0)),
                  pl.BlockSpec(memory_space=pltpu.SMEM)],
        out_specs=[pl.BlockSpec((nb, TN), lambda i: (0, 0)),
                   pl.BlockSpec((nb, TN), lambda i: (0, 0))],
        out_shape=[jax.ShapeDtypeStruct((nb, TN), jnp.float32),
                   jax.ShapeDtypeStruct((nb, TN), jnp.float32)],
    )
    xs1_r, xsw2_r = k1(xp, score_w1, score_w2, score_b1)

    eb_n = ek // be
    k2 = pl.pallas_call(
        _k2_body, grid=(eb_n,),
        in_specs=[pl.BlockSpec((DE, be), lambda i: (0, i)),
                  pl.BlockSpec((DE, 1), lambda i: (0, 0)),
                  pl.BlockSpec((DE, 1), lambda i: (0, 0)),
                  pl.BlockSpec(memory_space=pltpu.SMEM),
                  pl.BlockSpec(memory_space=pltpu.SMEM)],
        out_specs=[pl.BlockSpec((1, be), lambda i: (0, i)),
                   pl.BlockSpec((1, be), lambda i: (0, i))],
        out_shape=[jax.ShapeDtypeStruct((1, ek), jnp.float32),
                   jax.ShapeDtypeStruct((1, ek), jnp.float32)],
    )
    w1_c, w2_c = k2(eaT, dist_w1, dist_w2, dist_b1, dist_b2)
    w1 = w1_c.reshape(ek)
    w2 = w2_c.reshape(ek)
    if ep > ek:
        w1 = jnp.pad(w1, (0, ep - ek))
        w2 = jnp.pad(w2, (0, ep - ek))

    gidx = jnp.arange(G, dtype=batch_p.dtype)
    starts = jnp.searchsorted(batch_p[:N], gidx, side="left").astype(jnp.int32)
    ends = jnp.searchsorted(batch_p[:N], gidx, side="right").astype(jnp.int32)
    counts = (ends - starts).astype(jnp.float32).reshape(G, 1)
    tile0 = jnp.arange(nb, dtype=jnp.int32) * TN
    first_node = jnp.minimum(tile0, N - 1)
    last_node = jnp.minimum(tile0 + TN - 1, N - 1)
    g_first = batch_p[first_node]
    g_last = batch_p[last_node]
    lo = starts[g_first] // TN
    hi = jnp.maximum(ends[g_last] - 1, 0) // TN

    sc_edge = _make_sc_edge(epw, npad)
    r_call = _make_r(nb, npad)

    a1 = xs1_r.reshape(npad)
    b1 = jnp.ones((npad,), jnp.float32)
    agg1, cnt1 = sc_edge(ei, w1, a1, b1)
    alive1 = (jnp.arange(npad, dtype=jnp.int32) < N).astype(
        jnp.float32).reshape(nb, TN)
    zeros_row = jnp.zeros((nb, TN), jnp.float32)
    s1, sel1, a2r, b2r, c1 = r_call(
        agg1.reshape(NW, nb, TN), cnt1.reshape(NW, nb, TN),
        zeros_row, alive1, batchf, xsw2_r, counts, score_b2, lo, hi)

    agg2, cnt2 = sc_edge(ei, w2, a2r.reshape(npad), b2r.reshape(npad))
    s2, sel2, _, _, c2 = r_call(
        agg2.reshape(NW, nb, TN), cnt2.reshape(NW, nb, TN),
        s1, sel1, batchf, xsw2_r, c1, score_b2, lo, hi)

    f_call = _make_f(nb, npad, D)
    out = f_call(xp, batchf, s1, s2, sel1, sel2, counts, c1, c2)
    return out


# speedup vs baseline: 56.3449x; 1.6351x over previous
import functools

import jax
import jax.numpy as jnp
from jax import lax
from jax.experimental import pallas as pl
from jax.experimental.pallas import tpu as pltpu
from jax.experimental.pallas import tpu_sc as plsc

G = 64
RATIO_K = 0.5
TN = 128
NC = 2
NS = 16
NW = NC * NS


def _k1_body(tpb, x_ref, sw1_ref, sw2_ref, sb1_ref, xs1_ref, xsw2_ref):
    i = pl.program_id(0)
    xb = x_ref[...]
    dn = (((0,), (1,)), ((), ()))
    c1 = lax.dot_general(sw1_ref[...], xb, dn,
                         preferred_element_type=jnp.float32) + sb1_ref[0]
    c2 = lax.dot_general(sw2_ref[...], xb, dn,
                         preferred_element_type=jnp.float32)
    rows = pl.ds(i * tpb, tpb)
    xs1_ref[rows, :] = c1.reshape(tpb, TN)
    xsw2_ref[rows, :] = c2.reshape(tpb, TN)


def _k2_body(be, eaT_ref, dw1_ref, dw2_ref, db1_ref, db2_ref, w1_ref, w2_ref):
    i = pl.program_id(0)
    blk = eaT_ref[...]
    w1 = jnp.sum(blk * dw1_ref[...], axis=0) + db1_ref[0]
    w2 = jnp.sum(blk * dw2_ref[...], axis=0) + db2_ref[0]
    sl = pl.ds(i * be, be)
    w1_ref[sl] = w1
    w2_ref[sl] = w2


def _sc_edge_body(epw, npad, ei_hbm, w_hbm, a_hbm, b_hbm,
                  agg_out, cnt_out, ei_v, w_v, a_v, b_v, agg_v, cnt_v):
    wid = lax.axis_index("s") * NC + lax.axis_index("c")
    base = wid * epw
    pltpu.sync_copy(ei_hbm.at[:, pl.ds(base, epw)], ei_v)
    pltpu.sync_copy(w_hbm.at[pl.ds(base, epw)], w_v)
    pltpu.sync_copy(a_hbm, a_v)
    pltpu.sync_copy(b_hbm, b_v)

    zeros = jnp.zeros((16,), jnp.float32)

    def zbody(i, carry):
        agg_v[pl.ds(i * 16, 16)] = zeros
        cnt_v[pl.ds(i * 16, 16)] = zeros
        return carry

    lax.fori_loop(0, npad // 16, zbody, 0, unroll=4)

    def body(i, carry):
        sl = pl.ds(i * 16, 16)
        e0s = ei_v[0, sl]
        e1s = ei_v[1, sl]
        ws = w_v[sl]
        av = plsc.load_gather(a_v, [e0s])
        bv = plsc.load_gather(b_v, [e0s])
        plsc.addupdate_scatter(agg_v, [e1s], av * ws)
        plsc.addupdate_scatter(cnt_v, [e1s], bv)
        return carry

    lax.fori_loop(0, epw // 16, body, 0, unroll=4)

    pltpu.sync_copy(agg_v, agg_out.at[wid])
    pltpu.sync_copy(cnt_v, cnt_out.at[wid])


def _make_sc_edge(epw, npad):
    mesh = plsc.VectorSubcoreMesh(core_axis_name="c", subcore_axis_name="s",
                                  num_cores=NC, num_subcores=NS)
    return pl.kernel(
        functools.partial(_sc_edge_body, epw, npad),
        out_type=[jax.ShapeDtypeStruct((NW, npad), jnp.float32),
                  jax.ShapeDtypeStruct((NW, npad), jnp.float32)],
        mesh=mesh,
        compiler_params=pltpu.CompilerParams(needs_layout_passes=False),
        scratch_types=[pltpu.VMEM((2, epw), jnp.int32),
                       pltpu.VMEM((epw,), jnp.float32),
                       pltpu.VMEM((npad,), jnp.float32),
                       pltpu.VMEM((npad,), jnp.float32),
                       pltpu.VMEM((npad,), jnp.float32),
                       pltpu.VMEM((npad,), jnp.float32)],
    )


def _r_body(nb,
            pagg_ref, pcnt_ref, sec_ref, alive_ref, batchf_ref, xsw2_ref,
            nal_ref, sb2_ref, lo_ref, hi_ref,
            s_ref, sel_ref, an_ref, bn_ref, cg_ref,
            colS, colSec, colCode, rowS):
    def p0(i, carry):
        row = pl.ds(i, 1)
        agg = jnp.sum(pagg_ref[:, row, :], axis=0)
        cnt = jnp.sum(pcnt_ref[:, row, :], axis=0)
        s_row = agg / jnp.maximum(cnt, 1.0)
        s_ref[row, :] = s_row
        rowS[row, :] = s_row
        sl = pl.ds(i * TN, TN)
        colS[sl, :] = s_row.reshape(TN, 1)
        colSec[sl, :] = sec_ref[row, :].reshape(TN, 1)
        code = jnp.where(alive_ref[row, :] > 0.0, batchf_ref[row, :], -1.0)
        colCode[sl, :] = code.reshape(TN, 1)
        return carry

    lax.fori_loop(0, nb, p0, 0)

    kcol = jnp.ceil(RATIO_K * nal_ref[...])
    gio = lax.broadcasted_iota(jnp.int32, (G, 1), 0).astype(jnp.float32)
    iot_row = lax.broadcasted_iota(jnp.int32, (1, TN), 1).astype(jnp.float32)
    iot_col = lax.broadcasted_iota(jnp.int32, (TN, 1), 0).astype(jnp.float32)

    def p1(i, cg):
        row = pl.ds(i, 1)
        s_row = rowS[row, :]
        sec_row = sec_ref[row, :]
        b_row = batchf_ref[row, :]
        gp = (i * TN).astype(jnp.float32) + iot_row

        def body(j, rank):
            sl = pl.ds(j * TN, TN)
            sqc = colS[sl, :]
            secqc = colSec[sl, :]
            codeqc = colCode[sl, :]
            gq = (j * TN).astype(jnp.float32) + iot_col
            same = codeqc == b_row
            ahead = ((sqc > s_row)
                     | ((sqc == s_row)
                        & ((secqc > sec_row)
                           | ((secqc == sec_row) & (gq < gp)))))
            hit = jnp.where(same & ahead, 1.0, 0.0)
            return rank + jnp.sum(hit, axis=0, keepdims=True)

        rank = lax.fori_loop(lo_ref[i], hi_ref[i] + 1, body,
                             jnp.zeros((1, TN), jnp.float32))

        ohm = gio == b_row
        kp = jnp.sum(jnp.where(ohm, kcol, 0.0), axis=0, keepdims=True)
        selr = jnp.where((alive_ref[row, :] > 0.0) & (rank < kp), 1.0, 0.0)
        sel_ref[row, :] = selr
        t1 = jnp.tanh(s_row)
        an_ref[row, :] = selr * (t1 * xsw2_ref[row, :] + sb2_ref[0])
        bn_ref[row, :] = selr
        return cg + jnp.sum(jnp.where(ohm, selr, 0.0), axis=1, keepdims=True)

    cg = lax.fori_loop(0, nb, p1, jnp.zeros((G, 1), jnp.float32))
    cg_ref[...] = cg


def _make_r(nb, npad):
    vspec = pl.BlockSpec(memory_space=pltpu.VMEM)
    sspec = pl.BlockSpec(memory_space=pltpu.SMEM)
    return pl.pallas_call(
        functools.partial(_r_body, nb),
        in_specs=[vspec, vspec, vspec, vspec, vspec, vspec, vspec,
                  sspec, sspec, sspec],
        out_specs=[vspec, vspec, vspec, vspec, vspec],
        out_shape=[
            jax.ShapeDtypeStruct((nb, TN), jnp.float32),
            jax.ShapeDtypeStruct((nb, TN), jnp.float32),
            jax.ShapeDtypeStruct((nb, TN), jnp.float32),
            jax.ShapeDtypeStruct((nb, TN), jnp.float32),
            jax.ShapeDtypeStruct((G, 1), jnp.float32),
        ],
        scratch_shapes=[
            pltpu.VMEM((nb * TN, 1), jnp.float32),
            pltpu.VMEM((nb * TN, 1), jnp.float32),
            pltpu.VMEM((nb * TN, 1), jnp.float32),
            pltpu.VMEM((nb, TN), jnp.float32),
        ],
    )


def _f_body(nblk, tpb, x_ref, batchf_ref, s1_ref, s2_ref, sel1_ref, sel2_ref,
            cnt0_ref, c1_ref, c2_ref, out_ref, acc):
    i = pl.program_id(0)

    @pl.when(i == 0)
    def _init():
        acc[...] = jnp.zeros_like(acc)

    ic0 = 1.0 / jnp.maximum(cnt0_ref[...], 1.0)
    ic1 = 1.0 / jnp.maximum(c1_ref[...], 1.0)
    ic2 = 1.0 / jnp.maximum(c2_ref[...], 1.0)
    gio = lax.broadcasted_iota(jnp.int32, (G, 1), 0).astype(jnp.float32)

    a = acc[...]
    for sub in range(tpb):
        row = pl.ds(i * tpb + sub, 1)
        t1 = jnp.tanh(s1_ref[row, :])
        t2 = jnp.tanh(s2_ref[row, :])
        ohm = gio == batchf_ref[row, :]
        w1r = sel1_ref[row, :] * t1
        w2r = sel2_ref[row, :] * t1 * t2
        A = jnp.where(ohm, ic0 + w1r * ic1 + w2r * ic2, 0.0)
        a = a + jnp.dot(A, x_ref[pl.ds(sub * TN, TN), :],
                        preferred_element_type=jnp.float32)
    acc[...] = a

    @pl.when(i == nblk - 1)
    def _fin():
        out_ref[...] = acc[...]


def _make_f(nblk, tpb, d):
    vspec = pl.BlockSpec(memory_space=pltpu.VMEM)
    gcol = pl.BlockSpec((G, 1), lambda i: (0, 0))
    return pl.pallas_call(
        functools.partial(_f_body, nblk, tpb),
        grid=(nblk,),
        in_specs=[pl.BlockSpec((tpb * TN, d), lambda i: (i, 0)),
                  vspec, vspec, vspec, vspec, vspec,
                  gcol, gcol, gcol],
        out_specs=pl.BlockSpec((G, d), lambda i: (0, 0)),
        out_shape=jax.ShapeDtypeStruct((G, d), jnp.float32),
        scratch_shapes=[pltpu.VMEM((G, d), jnp.float32)],
    )


def kernel(x, edge_index, edge_attr, batch, score_w1, score_b1, dist_w1,
           dist_b1, score_w2, score_b2, dist_w2, dist_b2):
    N, D = x.shape
    E = edge_attr.shape[0]
    DE = edge_attr.shape[1]

    nb = -(-N // (TN * 8)) * 8
    npad = nb * TN
    tpb = nb // 8
    for be in (16000, 12800, 6400, 5120, 2560):
        if E % be == 0:
            break
    ek = -(-E // be) * be
    ep = -(-E // (NW * TN)) * (NW * TN)
    ep = max(ep, ek)
    epw = ep // NW

    xp = jnp.pad(x, ((0, npad - N), (0, 0)))
    batch_p = jnp.pad(batch.astype(jnp.int32), (0, npad - N),
                      constant_values=G - 1)
    batchf = batch_p.astype(jnp.float32).reshape(nb, TN)
    ei = edge_index.astype(jnp.int32)
    eaT = edge_attr.T
    if ep != E:
        pad1 = jnp.full((1, ep - E), 0, jnp.int32)
        pad2 = jnp.full((1, ep - E), npad - 1, jnp.int32)
        ei = jnp.concatenate(
            [ei, jnp.concatenate([pad1, pad2], axis=0)], axis=1)
    if ek != E:
        eaT = jnp.pad(eaT, ((0, 0), (0, ek - E)))

    k1 = pl.pallas_call(
        functools.partial(_k1_body, tpb), grid=(8,),
        in_specs=[pl.BlockSpec((tpb * TN, D), lambda i: (i, 0)),
                  pl.BlockSpec((D, 1), lambda i: (0, 0)),
                  pl.BlockSpec((D, 1), lambda i: (0, 0)),
                  pl.BlockSpec(memory_space=pltpu.SMEM)],
        out_specs=[pl.BlockSpec(memory_space=pltpu.VMEM),
                   pl.BlockSpec(memory_space=pltpu.VMEM)],
        out_shape=[jax.ShapeDtypeStruct((nb, TN), jnp.float32),
                   jax.ShapeDtypeStruct((nb, TN), jnp.float32)],
    )
    xs1_r, xsw2_r = k1(xp, score_w1, score_w2, score_b1)

    eb_n = ek // be
    k2 = pl.pallas_call(
        functools.partial(_k2_body, be), grid=(eb_n,),
        in_specs=[pl.BlockSpec((DE, be), lambda i: (0, i)),
                  pl.BlockSpec((DE, 1), lambda i: (0, 0)),
                  pl.BlockSpec((DE, 1), lambda i: (0, 0)),
                  pl.BlockSpec(memory_space=pltpu.SMEM),
                  pl.BlockSpec(memory_space=pltpu.SMEM)],
        out_specs=[pl.BlockSpec(memory_space=pltpu.VMEM),
                   pl.BlockSpec(memory_space=pltpu.VMEM)],
        out_shape=[jax.ShapeDtypeStruct((ek,), jnp.float32),
                   jax.ShapeDtypeStruct((ek,), jnp.float32)],
    )
    w1, w2 = k2(eaT, dist_w1, dist_w2, dist_b1, dist_b2)
    if ep > ek:
        w1 = jnp.pad(w1, (0, ep - ek))
        w2 = jnp.pad(w2, (0, ep - ek))

    gidx = jnp.arange(G, dtype=batch_p.dtype)
    starts = jnp.searchsorted(batch_p[:N], gidx, side="left").astype(jnp.int32)
    ends = jnp.searchsorted(batch_p[:N], gidx, side="right").astype(jnp.int32)
    counts = (ends - starts).astype(jnp.float32).reshape(G, 1)
    tile0 = jnp.arange(nb, dtype=jnp.int32) * TN
    first_node = jnp.minimum(tile0, N - 1)
    last_node = jnp.minimum(tile0 + TN - 1, N - 1)
    g_first = batch_p[first_node]
    g_last = batch_p[last_node]
    lo = starts[g_first] // TN
    hi = jnp.maximum(ends[g_last] - 1, 0) // TN

    sc_edge = _make_sc_edge(epw, npad)
    r_call = _make_r(nb, npad)

    a1 = xs1_r.reshape(npad)
    b1 = jnp.ones((npad,), jnp.float32)
    agg1, cnt1 = sc_edge(ei, w1, a1, b1)
    alive1 = (jnp.arange(npad, dtype=jnp.int32) < N).astype(
        jnp.float32).reshape(nb, TN)
    zeros_row = jnp.zeros((nb, TN), jnp.float32)
    s1, sel1, a2r, b2r, c1 = r_call(
        agg1.reshape(NW, nb, TN), cnt1.reshape(NW, nb, TN),
        zeros_row, alive1, batchf, xsw2_r, counts, score_b2, lo, hi)

    agg2, cnt2 = sc_edge(ei, w2, a2r.reshape(npad), b2r.reshape(npad))
    s2, sel2, _, _, c2 = r_call(
        agg2.reshape(NW, nb, TN), cnt2.reshape(NW, nb, TN),
        s1, sel1, batchf, xsw2_r, c1, score_b2, lo, hi)

    f_call = _make_f(8, tpb, D)
    out = f_call(xp, batchf, s1, s2, sel1, sel2, counts, c1, c2)
    return out
